# baseline (speedup 1.0000x reference)
import functools

import jax
import jax.numpy as jnp
from jax import lax
from jax.experimental import pallas as pl
from jax.experimental.pallas import tpu as pltpu
from jax.experimental.pallas import tpu_sc as plsc

MAX_APPLES = 1000
N_ACTIONS = 100
EMB_DIM = 64
BATCH = 16384
OUT_COLS = 1 + N_ACTIONS * EMB_DIM

NUM_WORKERS = 32
BPW = BATCH // NUM_WORKERS
NQ = BPW // 128
NG = BPW // 16


def _sc_body(xt_hbm, pair_hbm, last_hbm, out_hbm,
             xa0_v, xb0_v, xa1_v, xb1_v, idx0_v, idx1_v,
             rows0_v, rows1_v, last_v, tl_v,
             isem0, isem1, gsem0, gsem1, wsem0, wsem1):
    wid = lax.axis_index("s") * 2 + lax.axis_index("c")
    base = wid * BPW
    BANK = N_ACTIONS * N_ACTIONS

    xa = [xa0_v, xa1_v]
    xb = [xb0_v, xb1_v]
    idx = [idx0_v, idx1_v]
    rows = [rows0_v, rows1_v]
    isem = [isem0, isem1]
    gsem = [gsem0, gsem1]
    wsem = [wsem0, wsem1]

    def out_block(m, s):
        t = m >> 1
        h = pl.multiple_of((m & 1) * EMB_DIM, EMB_DIM)
        return out_hbm.at[pl.ds(base // 8, BPW // 8), t, s, pl.ds(h, EMB_DIM)]

    def icopy(m, b):
        pltpu.async_copy(xt_hbm.at[m, pl.ds(base, BPW)], xa[b], isem[b])
        pltpu.async_copy(xt_hbm.at[m + 1, pl.ds(base, BPW)], xb[b], isem[b])

    def iwait(b):
        for _ in range(2):
            pltpu.make_async_copy(
                xt_hbm.at[0, pl.ds(base, BPW)], xa[b], isem[b]
            ).wait()

    def compute(b, bank):
        for g in range(NG):
            p = lane16 + 16 * g
            r = 8 * lax.bitwise_and(p, 63) + lax.shift_right_logical(p, 6)
            a = plsc.load_gather(xa[b], [r])
            c = plsc.load_gather(xb[b], [r])
            idx[b][g // 8, pl.ds(16 * (g % 8), 16)] = a * N_ACTIONS + c + bank

    def gfire(b):
        for q in range(NQ):
            pltpu.async_copy(
                pair_hbm.at[idx[b].at[q]], rows[b].at[pl.ds(128 * q, 128)],
                gsem[b],
            )

    def gwait(b):
        for q in range(NQ):
            pltpu.make_async_copy(
                pair_hbm.at[idx[b].at[q]], rows[b].at[pl.ds(128 * q, 128)],
                gsem[b],
            ).wait()

    def wfire(m, b):
        for s in range(8):
            pltpu.async_copy(
                rows[b].at[pl.ds(64 * s, 64)], out_block(m, s), wsem[b]
            )

    def wdrain(b):
        for s in range(8):
            pltpu.make_async_copy(
                rows[b].at[pl.ds(64 * s, 64)], out_block(0, s), wsem[b]
            ).wait()

    pltpu.sync_copy(last_hbm, tl_v)
    pltpu.sync_copy(xt_hbm.at[N_ACTIONS, pl.ds(base, BPW)], xb0_v)
    zero16 = jnp.zeros((16,), jnp.int32)
    lane16 = lax.iota(jnp.int32, 16)
    for g in range(NG):
        r = lane16 + 16 * g
        c = xb0_v[pl.ds(16 * g, 16)]
        plsc.store_scatter(
            last_v,
            [lax.shift_right_logical(r, 3), lax.bitwise_and(r, 7), zero16],
            plsc.load_gather(tl_v, [c]),
        )
    pltpu.sync_copy(
        last_v,
        out_hbm.at[
            pl.ds(base // 8, BPW // 8), (OUT_COLS - 1) // 128, :, pl.ds(0, 1)
        ],
    )

    icopy(0, 0)
    iwait(0)
    compute(0, 0)
    icopy(1, 1)
    gfire(0)

    def step(m, b):
        iwait(1 - b)
        compute(1 - b, BANK)
        icopy(m + 2, b)
        wdrain(1 - b)
        gfire(1 - b)
        gwait(b)
        wfire(m, b)

    iwait(1)
    compute(1, BANK)
    icopy(2, 0)
    gfire(1)
    gwait(0)
    wfire(0, 0)

    def body(k, carry):
        step(2 * k + 1, 1)
        step(2 * k + 2, 0)
        return carry

    lax.fori_loop(0, 48, body, 0)

    iwait(0)
    compute(0, BANK)
    wdrain(0)
    gfire(0)
    gwait(1)
    wfire(97, 1)
    icopy(99, 1)
    iwait(1)
    compute(1, BANK)
    wdrain(1)
    gfire(1)
    gwait(0)
    wfire(98, 0)
    gwait(1)
    wfire(99, 1)
    wdrain(0)
    wdrain(1)


@jax.jit
def _sc_call(xt, pair, last):
    mesh = plsc.VectorSubcoreMesh(core_axis_name="c", subcore_axis_name="s")
    f = functools.partial(
        pl.kernel,
        mesh=mesh,
        out_type=jax.ShapeDtypeStruct(
            (BATCH // 8, (OUT_COLS + 127) // 128, 8, 128), jnp.float32
        ),
        scratch_types=[
            pltpu.VMEM((BPW,), jnp.int32),
            pltpu.VMEM((BPW,), jnp.int32),
            pltpu.VMEM((BPW,), jnp.int32),
            pltpu.VMEM((BPW,), jnp.int32),
            pltpu.VMEM((NQ, 128), jnp.int32),
            pltpu.VMEM((NQ, 128), jnp.int32),
            pltpu.VMEM((BPW, EMB_DIM), jnp.float32),
            pltpu.VMEM((BPW, EMB_DIM), jnp.float32),
            pltpu.VMEM((BPW // 8, 8, 1), jnp.float32),
            pltpu.VMEM((N_ACTIONS,), jnp.float32),
            pltpu.SemaphoreType.DMA,
            pltpu.SemaphoreType.DMA,
            pltpu.SemaphoreType.DMA,
            pltpu.SemaphoreType.DMA,
            pltpu.SemaphoreType.DMA,
            pltpu.SemaphoreType.DMA,
        ],
        compiler_params=pltpu.CompilerParams(
            use_tc_tiling_on_sc=False, needs_layout_passes=False
        ),
    )(_sc_body)
    return f(xt, pair, last)


def kernel(x, emb_table):
    xt = x.T
    lut = 2.0 * jnp.arange(N_ACTIONS, dtype=jnp.float32) / MAX_APPLES - 1.0
    rest = jnp.broadcast_to(
        emb_table[None, :, : EMB_DIM - 1], (N_ACTIONS, N_ACTIONS, EMB_DIM - 1)
    )
    p_lut = jnp.concatenate(
        [jnp.broadcast_to(lut[:, None, None], (N_ACTIONS, N_ACTIONS, 1)), rest],
        axis=-1,
    )
    p_emb = jnp.concatenate(
        [
            jnp.broadcast_to(
                emb_table[:, None, EMB_DIM - 1 :], (N_ACTIONS, N_ACTIONS, 1)
            ),
            rest,
        ],
        axis=-1,
    )
    pair = jnp.concatenate(
        [
            p_lut.reshape(N_ACTIONS * N_ACTIONS, EMB_DIM),
            p_emb.reshape(N_ACTIONS * N_ACTIONS, EMB_DIM),
        ],
        axis=0,
    )
    last = emb_table[:, EMB_DIM - 1]
    tiled = _sc_call(xt, pair, last)
    return _tc_relayout(tiled)


def _tc_body(l4_ref, out_ref):
    out_ref[...] = l4_ref[:, 0, :, :].reshape(BATCH, 128)


@jax.jit
def _tc_relayout(l4):
    n_tiles = (OUT_COLS + 127) // 128
    return pl.pallas_call(
        _tc_body,
        grid=(n_tiles,),
        in_specs=[
            pl.BlockSpec((BATCH // 8, 1, 8, 128), lambda i: (0, i, 0, 0))
        ],
        out_specs=pl.BlockSpec((BATCH, 128), lambda i: (0, i)),
        out_shape=jax.ShapeDtypeStruct((BATCH, OUT_COLS), jnp.float32),
    )(l4)


# speedup vs baseline: 1.1435x; 1.1435x over previous
import functools

import jax
import jax.numpy as jnp
from jax import lax
from jax.experimental import pallas as pl
from jax.experimental.pallas import tpu as pltpu
from jax.experimental.pallas import tpu_sc as plsc

MAX_APPLES = 1000
N_ACTIONS = 100
EMB_DIM = 64
BATCH = 16384
OUT_COLS = 1 + N_ACTIONS * EMB_DIM
N_TILES = N_ACTIONS * EMB_DIM // 128

NUM_WORKERS = 32
BPW = BATCH // NUM_WORKERS
SUB = 64
NSUB = BPW // SUB
BANK = N_ACTIONS * N_ACTIONS


def _sc_body(xt_hbm, pair_hbm, last_hbm, out_hbm,
             xc0, xc1, xc2, xn0, xn1, xn2, idxa_v, idxb_v,
             ra0, ra1, ra2, ra3, rb0, rb1, rb2, rb3,
             mg0, mg1, last_v, tl_v, isem, gsem, wsem0, wsem1):
    wid = lax.axis_index("s") * 2 + lax.axis_index("c")
    base = wid * BPW
    ra = [ra0, ra1, ra2, ra3]
    rb = [rb0, rb1, rb2, rb3]
    mg = [mg0, mg1]
    lane16 = lax.iota(jnp.int32, 16)
    zero16 = jnp.zeros((16,), jnp.int32)

    def icopy(t):
        pltpu.async_copy(xt_hbm.at[2 * t, :, pl.ds(base, BPW)], xn0, isem)
        pltpu.async_copy(xt_hbm.at[2 * t + 1, :, pl.ds(base, BPW)], xn1, isem)
        pltpu.async_copy(xt_hbm.at[2 * t + 2, :, pl.ds(base, BPW)], xn2, isem)

    def iwait():
        for _ in range(3):
            pltpu.make_async_copy(
                xt_hbm.at[0, :, pl.ds(base, BPW)], xn0, isem
            ).wait()

    def out_tile(t, u):
        col = pl.multiple_of(128 * t, 128)
        return out_hbm.at[pl.ds(base + SUB * u, SUB), pl.ds(col, 128)]

    def gdesc(u):
        return (
            pltpu.make_async_copy(pair_hbm.at[idxa_v.at[u]], ra[u % 4], gsem),
            pltpu.make_async_copy(pair_hbm.at[idxb_v.at[u]], rb[u % 4], gsem),
        )

    wsem = [wsem0, wsem1]

    def wdesc(u):
        return pltpu.make_async_copy(mg[u % 2], out_tile(0, 0), wsem[u % 2])

    pltpu.sync_copy(xt_hbm.at[0, :, pl.ds(base, BPW)], xc0)
    pltpu.sync_copy(xt_hbm.at[1, :, pl.ds(base, BPW)], xc1)
    pltpu.sync_copy(xt_hbm.at[2, :, pl.ds(base, BPW)], xc2)
    icopy(1)

    def tile_body(t, carry):
        bank_a = jnp.where(t == 0, 0, BANK).astype(jnp.int32)
        for u in range(NSUB):
            for g in range(SUB // 16):
                o = SUB * u + 16 * g
                a = xc0[0, pl.ds(o, 16)]
                b = xc1[0, pl.ds(o, 16)]
                c = xc2[0, pl.ds(o, 16)]
                idxa_v[u, pl.ds(16 * g, 16)] = a * N_ACTIONS + b + bank_a
                idxb_v[u, pl.ds(16 * g, 16)] = b * N_ACTIONS + c + BANK
        for u in range(4):
            da, db = gdesc(u)
            da.start()
            db.start()
        iwait()
        for g in range(BPW // 16):
            s = pl.ds(16 * g, 16)
            xc0[0, s] = xn0[0, s]
            xc1[0, s] = xn1[0, s]
            xc2[0, s] = xn2[0, s]
        icopy(jnp.minimum(t + 2, N_TILES - 1))
        for u in range(NSUB):
            da, db = gdesc(u)
            da.wait()
            db.wait()
            if u < 2:
                @pl.when(t > 0)
                def _():
                    wdesc(u).wait()
            else:
                wdesc(u).wait()

            def merge_row(r, carry2):
                for j in range(4):
                    mg[u % 2][r, pl.ds(16 * j, 16)] = (
                        ra[u % 4][r, pl.ds(16 * j, 16)]
                    )
                    mg[u % 2][r, pl.ds(64 + 16 * j, 16)] = (
                        rb[u % 4][r, pl.ds(16 * j, 16)]
                    )
                return carry2

            lax.fori_loop(0, SUB, merge_row, 0)
            pltpu.async_copy(mg[u % 2], out_tile(t, u), wsem[u % 2])
            if u + 4 < NSUB:
                da2, db2 = gdesc(u + 4)
                da2.start()
                db2.start()
        return carry

    lax.fori_loop(0, N_TILES, tile_body, 0)
    wdesc(0).wait()
    wdesc(1).wait()
    iwait()

    pltpu.sync_copy(last_hbm, tl_v)
    pltpu.sync_copy(xt_hbm.at[N_ACTIONS, :, pl.ds(base, BPW)], xc0)
    for u in range(NSUB):
        for g in range(SUB // 16):
            c = xc0[0, pl.ds(SUB * u + 16 * g, 16)]
            plsc.store_scatter(last_v, [lane16 + 16 * g, zero16],
                               plsc.load_gather(tl_v, [c]))
        pltpu.sync_copy(
            last_v,
            out_hbm.at[pl.ds(base + SUB * u, SUB), pl.ds(OUT_COLS - 1, 1)],
        )


@jax.jit
def _sc_call(xt, pair, last):
    mesh = plsc.VectorSubcoreMesh(core_axis_name="c", subcore_axis_name="s")
    f = functools.partial(
        pl.kernel,
        mesh=mesh,
        out_type=jax.ShapeDtypeStruct((BATCH, OUT_COLS), jnp.float32),
        scratch_types=[
            pltpu.VMEM((1, BPW), jnp.int32),
            pltpu.VMEM((1, BPW), jnp.int32),
            pltpu.VMEM((1, BPW), jnp.int32),
            pltpu.VMEM((1, BPW), jnp.int32),
            pltpu.VMEM((1, BPW), jnp.int32),
            pltpu.VMEM((1, BPW), jnp.int32),
            pltpu.VMEM((NSUB, SUB), jnp.int32),
            pltpu.VMEM((NSUB, SUB), jnp.int32),
            pltpu.VMEM((SUB, 128), jnp.float32),
            pltpu.VMEM((SUB, 128), jnp.float32),
            pltpu.VMEM((SUB, 128), jnp.float32),
            pltpu.VMEM((SUB, 128), jnp.float32),
            pltpu.VMEM((SUB, 128), jnp.float32),
            pltpu.VMEM((SUB, 128), jnp.float32),
            pltpu.VMEM((SUB, 128), jnp.float32),
            pltpu.VMEM((SUB, 128), jnp.float32),
            pltpu.VMEM((SUB, 128), jnp.float32),
            pltpu.VMEM((SUB, 128), jnp.float32),
            pltpu.VMEM((SUB, 1), jnp.float32),
            pltpu.VMEM((N_ACTIONS,), jnp.float32),
            pltpu.SemaphoreType.DMA,
            pltpu.SemaphoreType.DMA,
            pltpu.SemaphoreType.DMA,
            pltpu.SemaphoreType.DMA,
        ],
        compiler_params=pltpu.CompilerParams(
            use_tc_tiling_on_sc=True, needs_layout_passes=False
        ),
    )(_sc_body)
    return f(xt, pair, last)


def kernel(x, emb_table):
    xt = x.T.reshape(1 + N_ACTIONS, 1, BATCH)
    lut = 2.0 * jnp.arange(N_ACTIONS, dtype=jnp.float32) / MAX_APPLES - 1.0
    rest = jnp.broadcast_to(
        emb_table[None, :, : EMB_DIM - 1], (N_ACTIONS, N_ACTIONS, EMB_DIM - 1)
    )
    p_lut = jnp.concatenate(
        [jnp.broadcast_to(lut[:, None, None], (N_ACTIONS, N_ACTIONS, 1)), rest],
        axis=-1,
    )
    p_emb = jnp.concatenate(
        [
            jnp.broadcast_to(
                emb_table[:, None, EMB_DIM - 1 :], (N_ACTIONS, N_ACTIONS, 1)
            ),
            rest,
        ],
        axis=-1,
    )
    pair = jnp.concatenate(
        [
            p_lut.reshape(N_ACTIONS * N_ACTIONS, EMB_DIM),
            p_emb.reshape(N_ACTIONS * N_ACTIONS, EMB_DIM),
        ],
        axis=0,
    )
    pair = jnp.pad(pair, ((0, 0), (0, 128 - EMB_DIM)))
    last = emb_table[:, EMB_DIM - 1]
    return _sc_call(xt, pair, last)


# speedup vs baseline: 1.5038x; 1.3151x over previous
import functools

import jax
import jax.numpy as jnp
from jax import lax
from jax.experimental import pallas as pl
from jax.experimental.pallas import tpu as pltpu
from jax.experimental.pallas import tpu_sc as plsc

MAX_APPLES = 1000
N_ACTIONS = 100
EMB_DIM = 64
BATCH = 16384
OUT_COLS = 1 + N_ACTIONS * EMB_DIM

NUM_WORKERS = 32
BPW = BATCH // NUM_WORKERS
NQ = BPW // 128
NG = BPW // 16


def _sc_body(xt_hbm, pair_hbm, last_hbm, out_hbm,
             xa0_v, xb0_v, xa1_v, xb1_v, idx0_v, idx1_v,
             rows0_v, rows1_v, last_v, tl_v,
             isem0, isem1, gsem0, gsem1, wsem0, wsem1):
    wid = lax.axis_index("s") * 2 + lax.axis_index("c")
    base = wid * BPW
    BANK = N_ACTIONS * N_ACTIONS

    xa = [xa0_v, xa1_v]
    xb = [xb0_v, xb1_v]
    idx = [idx0_v, idx1_v]
    rows = [rows0_v, rows1_v]
    isem = [isem0, isem1]
    gsem = [gsem0, gsem1]
    wsem = [wsem0, wsem1]

    def out_block(m, s):
        t = m >> 1
        h = pl.multiple_of((m & 1) * EMB_DIM, EMB_DIM)
        return out_hbm.at[pl.ds(base // 8, BPW // 8), t, s, pl.ds(h, EMB_DIM)]

    def icopy(m, b):
        pltpu.async_copy(xt_hbm.at[m, pl.ds(base, BPW)], xa[b], isem[b])
        pltpu.async_copy(xt_hbm.at[m + 1, pl.ds(base, BPW)], xb[b], isem[b])

    def iwait(b):
        for _ in range(2):
            pltpu.make_async_copy(
                xt_hbm.at[0, pl.ds(base, BPW)], xa[b], isem[b]
            ).wait()

    def compute(b, bank):
        for g in range(NG):
            p = lane16 + 16 * g
            r = 8 * lax.bitwise_and(p, 63) + lax.shift_right_logical(p, 6)
            a = plsc.load_gather(xa[b], [r])
            c = plsc.load_gather(xb[b], [r])
            idx[b][g // 8, pl.ds(16 * (g % 8), 16)] = a * N_ACTIONS + c + bank

    def gfire(b):
        for q in range(NQ):
            pltpu.async_copy(
                pair_hbm.at[idx[b].at[q]], rows[b].at[pl.ds(128 * q, 128)],
                gsem[b],
            )

    def gwait(b):
        for q in range(NQ):
            pltpu.make_async_copy(
                pair_hbm.at[idx[b].at[q]], rows[b].at[pl.ds(128 * q, 128)],
                gsem[b],
            ).wait()

    def wfire(m, b):
        for s in range(8):
            pltpu.async_copy(
                rows[b].at[pl.ds(64 * s, 64)], out_block(m, s), wsem[b]
            )

    def wdrain(b):
        for s in range(8):
            pltpu.make_async_copy(
                rows[b].at[pl.ds(64 * s, 64)], out_block(0, s), wsem[b]
            ).wait()

    pltpu.sync_copy(last_hbm, tl_v)
    pltpu.sync_copy(xt_hbm.at[N_ACTIONS, pl.ds(base, BPW)], xb0_v)
    zero16 = jnp.zeros((16,), jnp.int32)
    lane16 = lax.iota(jnp.int32, 16)
    for g in range(NG):
        r = lane16 + 16 * g
        c = xb0_v[pl.ds(16 * g, 16)]
        plsc.store_scatter(
            last_v,
            [lax.shift_right_logical(r, 3), lax.bitwise_and(r, 7), zero16],
            plsc.load_gather(tl_v, [c]),
        )
    pltpu.sync_copy(
        last_v,
        out_hbm.at[
            pl.ds(base // 8, BPW // 8), (OUT_COLS - 1) // 128, :, pl.ds(0, 1)
        ],
    )

    icopy(0, 0)
    iwait(0)
    compute(0, 0)
    icopy(1, 1)
    gfire(0)

    def step(m, b):
        iwait(1 - b)
        compute(1 - b, BANK)
        icopy(m + 2, b)
        wdrain(1 - b)
        gfire(1 - b)
        gwait(b)
        wfire(m, b)

    iwait(1)
    compute(1, BANK)
    icopy(2, 0)
    gfire(1)
    gwait(0)
    wfire(0, 0)

    def body(k, carry):
        step(2 * k + 1, 1)
        step(2 * k + 2, 0)
        return carry

    lax.fori_loop(0, 48, body, 0)

    iwait(0)
    compute(0, BANK)
    wdrain(0)
    gfire(0)
    gwait(1)
    wfire(97, 1)
    icopy(99, 1)
    iwait(1)
    compute(1, BANK)
    wdrain(1)
    gfire(1)
    gwait(0)
    wfire(98, 0)
    gwait(1)
    wfire(99, 1)
    wdrain(0)
    wdrain(1)


@jax.jit
def _sc_call(xt, pair, last):
    mesh = plsc.VectorSubcoreMesh(core_axis_name="c", subcore_axis_name="s")
    f = functools.partial(
        pl.kernel,
        mesh=mesh,
        out_type=jax.ShapeDtypeStruct(
            (BATCH // 8, (OUT_COLS + 127) // 128, 8, 128), jnp.float32
        ),
        scratch_types=[
            pltpu.VMEM((BPW,), jnp.int32),
            pltpu.VMEM((BPW,), jnp.int32),
            pltpu.VMEM((BPW,), jnp.int32),
            pltpu.VMEM((BPW,), jnp.int32),
            pltpu.VMEM((NQ, 128), jnp.int32),
            pltpu.VMEM((NQ, 128), jnp.int32),
            pltpu.VMEM((BPW, EMB_DIM), jnp.float32),
            pltpu.VMEM((BPW, EMB_DIM), jnp.float32),
            pltpu.VMEM((BPW // 8, 8, 1), jnp.float32),
            pltpu.VMEM((N_ACTIONS,), jnp.float32),
            pltpu.SemaphoreType.DMA,
            pltpu.SemaphoreType.DMA,
            pltpu.SemaphoreType.DMA,
            pltpu.SemaphoreType.DMA,
            pltpu.SemaphoreType.DMA,
            pltpu.SemaphoreType.DMA,
        ],
        compiler_params=pltpu.CompilerParams(
            use_tc_tiling_on_sc=False, needs_layout_passes=False
        ),
    )(_sc_body)
    return f(xt, pair, last)


def kernel(x, emb_table):
    xt = x.T
    lut = 2.0 * jnp.arange(N_ACTIONS, dtype=jnp.float32) / MAX_APPLES - 1.0
    rest = jnp.broadcast_to(
        emb_table[None, :, : EMB_DIM - 1], (N_ACTIONS, N_ACTIONS, EMB_DIM - 1)
    )
    p_lut = jnp.concatenate(
        [jnp.broadcast_to(lut[:, None, None], (N_ACTIONS, N_ACTIONS, 1)), rest],
        axis=-1,
    )
    p_emb = jnp.concatenate(
        [
            jnp.broadcast_to(
                emb_table[:, None, EMB_DIM - 1 :], (N_ACTIONS, N_ACTIONS, 1)
            ),
            rest,
        ],
        axis=-1,
    )
    pair = jnp.concatenate(
        [
            p_lut.reshape(N_ACTIONS * N_ACTIONS, EMB_DIM),
            p_emb.reshape(N_ACTIONS * N_ACTIONS, EMB_DIM),
        ],
        axis=0,
    )
    last = emb_table[:, EMB_DIM - 1]
    tiled = _sc_call(xt, pair, last)
    wide = jnp.transpose(tiled, (0, 2, 1, 3)).reshape(BATCH, -1)
    return wide[:, :OUT_COLS]
